# baseline (device time: 174325 ns/iter reference)
import jax
import jax.numpy as jnp
from jax import lax
from jax.experimental import pallas as pl
from jax.experimental.pallas import tpu as pltpu

N_DEV = 4
B_LOC = 2
SQ = 512
SKV = 512
H_LOC = 8
DH = 64
D_MODEL = 768
D_BLOCK = H_LOC * DH
HWQ = D_MODEL // 2
HWO = D_BLOCK // 2

_CompilerParams = getattr(pltpu, "CompilerParams", None) or getattr(
    pltpu, "TPUCompilerParams"
)


def _body(x_ref, wq_ref, k_hbm, v_hbm, wo_ref, out_ref,
          x16, wq16, wo16, wq_buf, wo_buf, k_scr, v_scr, ctx_scr,
          ss, rs, k_sems, v_sems):
    my = lax.axis_index("i")
    left = (my - 1) % N_DEV
    right = (my + 1) % N_DEV

    def rcopy(src, dst, s_idx, r_idx, dev):
        return pltpu.make_async_remote_copy(
            src_ref=src, dst_ref=dst,
            send_sem=ss.at[s_idx], recv_sem=rs.at[r_idx],
            device_id=(dev,), device_id_type=pl.DeviceIdType.MESH,
        )

    hbs = [my, left, right, (my + 2) % N_DEV]
    bg = my * B_LOC

    kv_dmas = {}

    def issue_step(s):
        ds = []
        for h in range(H_LOC):
            head = hbs[s] * H_LOC + h
            i = s * H_LOC + h
            kd = pltpu.make_async_copy(
                k_hbm.at[pl.ds(bg, B_LOC), :, head, :],
                k_scr.at[s, h], k_sems.at[i])
            vd = pltpu.make_async_copy(
                v_hbm.at[pl.ds(bg, B_LOC), :, head, :],
                v_scr.at[s, h], v_sems.at[i])
            kd.start()
            vd.start()
            ds.append((kd, vd))
        kv_dmas[s] = ds

    issue_step(0)

    x16[...] = x_ref[...].astype(jnp.bfloat16)
    wq16[...] = wq_ref[...].astype(jnp.bfloat16)
    wo16[...] = wo_ref[...].astype(jnp.bfloat16)

    bar = pltpu.get_barrier_semaphore()
    pl.semaphore_signal(bar, inc=1, device_id=(left,),
                        device_id_type=pl.DeviceIdType.MESH)
    pl.semaphore_signal(bar, inc=1, device_id=(right,),
                        device_id_type=pl.DeviceIdType.MESH)
    pl.semaphore_wait(bar, 2)

    sends = [
        rcopy(wq16, wq_buf.at[0], 0, 0, right),
        rcopy(wo16, wo_buf.at[0], 1, 1, right),
        rcopy(wq16, wq_buf.at[1], 2, 2, left),
        rcopy(wo16, wo_buf.at[1], 3, 3, left),
    ]
    for r in sends:
        r.start()

    qi = lax.broadcasted_iota(jnp.int32, (SQ, SKV), 0)
    ki = lax.broadcasted_iota(jnp.int32, (SQ, SKV), 1)
    mask = (jnp.abs(qi - ki) <= 128) | (ki < 32) | (qi < 32)
    bias = jnp.where(mask, 0.0, -1e9).astype(jnp.float32)

    def compute(wq_src, wo_src, s, first):
        if s + 1 < N_DEV:
            issue_step(s + 1)
        for kd, vd in kv_dmas[s]:
            kd.wait()
            vd.wait()
        wq_s = wq_src[...]
        wo_s = wo_src[...]
        for b in range(B_LOC):
            qb = jnp.dot(x16[b], wq_s,
                         preferred_element_type=jnp.float32,
                         ).astype(jnp.bfloat16)
            for h in range(H_LOC):
                qh = qb[:, h * DH:(h + 1) * DH]
                kh = k_scr[s, h, b].astype(jnp.bfloat16)
                sc = lax.dot_general(
                    qh, kh, (((1,), (1,)), ((), ())),
                    preferred_element_type=jnp.float32)
                e = jnp.exp(sc * 0.125 + bias).astype(jnp.bfloat16)
                r = 1.0 / jnp.sum(e, axis=1, keepdims=True,
                                  dtype=jnp.float32)
                vh = v_scr[s, h, b].astype(jnp.bfloat16)
                ctxh = jnp.dot(e, vh,
                               preferred_element_type=jnp.float32)
                ctx_scr[:, h * DH:(h + 1) * DH] = (
                    ctxh * r).astype(jnp.bfloat16)
            contrib = jnp.dot(ctx_scr[...], wo_s,
                              preferred_element_type=jnp.float32)
            if first:
                out_ref[b] = contrib
            else:
                out_ref[b] = out_ref[b] + contrib

    compute(wq16, wo16, 0, first=True)

    for r in sends:
        r.wait_recv()
    fwds = [
        rcopy(wq_buf.at[0, pl.ds(0, HWQ), :], wq_buf.at[2, pl.ds(0, HWQ), :],
              4, 4, right),
        rcopy(wo_buf.at[0, pl.ds(0, HWO), :], wo_buf.at[2, pl.ds(0, HWO), :],
              5, 5, right),
        rcopy(wq_buf.at[1, pl.ds(HWQ, HWQ), :], wq_buf.at[2, pl.ds(HWQ, HWQ), :],
              6, 6, left),
        rcopy(wo_buf.at[1, pl.ds(HWO, HWO), :], wo_buf.at[2, pl.ds(HWO, HWO), :],
              7, 7, left),
    ]
    for r in fwds:
        r.start()

    compute(wq_buf.at[0], wo_buf.at[0], 1, first=False)
    compute(wq_buf.at[1], wo_buf.at[1], 2, first=False)

    for r in fwds:
        r.wait_recv()
    compute(wq_buf.at[2], wo_buf.at[2], 3, first=False)

    for r in sends + fwds:
        r.wait_send()


def kernel(x, Wq, K_ext, V_ext, Wo):
    return pl.pallas_call(
        _body,
        out_shape=jax.ShapeDtypeStruct((B_LOC, SQ, D_MODEL), jnp.float32),
        in_specs=[
            pl.BlockSpec(memory_space=pltpu.VMEM),
            pl.BlockSpec(memory_space=pltpu.VMEM),
            pl.BlockSpec(memory_space=pltpu.HBM),
            pl.BlockSpec(memory_space=pltpu.HBM),
            pl.BlockSpec(memory_space=pltpu.VMEM),
        ],
        out_specs=pl.BlockSpec(memory_space=pltpu.VMEM),
        scratch_shapes=[
            pltpu.VMEM((B_LOC, SQ, D_MODEL), jnp.bfloat16),
            pltpu.VMEM((D_MODEL, D_BLOCK), jnp.bfloat16),
            pltpu.VMEM((D_BLOCK, D_MODEL), jnp.bfloat16),
            pltpu.VMEM((N_DEV - 1, D_MODEL, D_BLOCK), jnp.bfloat16),
            pltpu.VMEM((N_DEV - 1, D_BLOCK, D_MODEL), jnp.bfloat16),
            pltpu.VMEM((N_DEV, H_LOC, B_LOC, SKV, DH), jnp.float32),
            pltpu.VMEM((N_DEV, H_LOC, B_LOC, SKV, DH), jnp.float32),
            pltpu.VMEM((SQ, D_BLOCK), jnp.bfloat16),
            pltpu.SemaphoreType.DMA((8,)),
            pltpu.SemaphoreType.DMA((8,)),
            pltpu.SemaphoreType.DMA((N_DEV * H_LOC,)),
            pltpu.SemaphoreType.DMA((N_DEV * H_LOC,)),
        ],
        compiler_params=_CompilerParams(
            collective_id=0, vmem_limit_bytes=100 * 1024 * 1024),
    )(x, Wq, K_ext, V_ext, Wo)


# device time: 73796 ns/iter; 2.3623x vs baseline; 2.3623x over previous
import jax
import jax.numpy as jnp
from jax import lax
from jax.experimental import pallas as pl
from jax.experimental.pallas import tpu as pltpu

N_DEV = 4
B_LOC = 2
SQ = 512
SKV = 512
H_LOC = 8
DH = 64
D_MODEL = 768
D_BLOCK = H_LOC * DH
HWQ = D_MODEL // 2
HWO = D_BLOCK // 2

_CompilerParams = getattr(pltpu, "CompilerParams", None) or getattr(
    pltpu, "TPUCompilerParams"
)


def _body(x_ref, wq_ref, k_ref, v_ref, wo_ref, out_ref,
          x16, wq16, wo16, wq_buf, wo_buf, ctx_scr, ss, rs):
    my = lax.axis_index("i")
    left = (my - 1) % N_DEV
    right = (my + 1) % N_DEV

    def rcopy(src, dst, s_idx, r_idx, dev):
        return pltpu.make_async_remote_copy(
            src_ref=src, dst_ref=dst,
            send_sem=ss.at[s_idx], recv_sem=rs.at[r_idx],
            device_id=(dev,), device_id_type=pl.DeviceIdType.MESH,
        )

    x16[...] = x_ref[...].astype(jnp.bfloat16)
    wq16[...] = wq_ref[...].astype(jnp.bfloat16)
    wo16[...] = wo_ref[...].astype(jnp.bfloat16)

    bar = pltpu.get_barrier_semaphore()
    pl.semaphore_signal(bar, inc=1, device_id=(left,),
                        device_id_type=pl.DeviceIdType.MESH)
    pl.semaphore_signal(bar, inc=1, device_id=(right,),
                        device_id_type=pl.DeviceIdType.MESH)
    pl.semaphore_wait(bar, 2)

    sends = [
        rcopy(wq16, wq_buf.at[0], 0, 0, right),
        rcopy(wq16, wq_buf.at[1], 2, 2, left),
        rcopy(wo16, wo_buf.at[0], 1, 1, right),
        rcopy(wo16, wo_buf.at[1], 3, 3, left),
    ]
    for r in sends:
        r.start()

    qi = lax.broadcasted_iota(jnp.int32, (SQ, SKV), 0)
    ki = lax.broadcasted_iota(jnp.int32, (SQ, SKV), 1)
    mask = (jnp.abs(qi - ki) <= 128) | (ki < 32) | (qi < 32)
    bias = jnp.where(mask, 0.0, -1e9).astype(jnp.float32)

    def compute(wq_src, wo_src, hb, first, wo_wait=None):
        wq_s = wq_src[...]
        wo_s = None
        for b in range(B_LOC):
            qb = jnp.dot(x16[b], wq_s,
                         preferred_element_type=jnp.float32,
                         ).astype(jnp.bfloat16)
            for h in range(H_LOC):
                head = hb * H_LOC + h
                qh = qb[:, h * DH:(h + 1) * DH]
                kh = k_ref[head, b]
                sc = lax.dot_general(
                    qh, kh, (((1,), (1,)), ((), ())),
                    preferred_element_type=jnp.float32)
                e = jnp.exp(sc * 0.125 + bias).astype(jnp.bfloat16)
                r = 1.0 / jnp.sum(e, axis=1, keepdims=True,
                                  dtype=jnp.float32)
                vh = v_ref[head, b]
                ctxh = jnp.dot(e, vh,
                               preferred_element_type=jnp.float32)
                ctx_scr[:, h * DH:(h + 1) * DH] = (
                    ctxh * r).astype(jnp.bfloat16)
            if wo_s is None:
                if wo_wait is not None:
                    wo_wait()
                wo_s = wo_src[...]
            contrib = jnp.dot(ctx_scr[...], wo_s,
                              preferred_element_type=jnp.float32)
            if first:
                out_ref[b] = contrib
            else:
                out_ref[b] = out_ref[b] + contrib

    compute(wq16, wo16, my, first=True)

    sends[0].wait_recv()
    sends[1].wait_recv()
    fwds_wq = [
        rcopy(wq_buf.at[0, pl.ds(0, HWQ), :], wq_buf.at[2, pl.ds(0, HWQ), :],
              4, 4, right),
        rcopy(wq_buf.at[1, pl.ds(HWQ, HWQ), :], wq_buf.at[2, pl.ds(HWQ, HWQ), :],
              6, 6, left),
    ]
    for r in fwds_wq:
        r.start()

    fwds_wo = [
        rcopy(wo_buf.at[0, pl.ds(0, HWO), :], wo_buf.at[2, pl.ds(0, HWO), :],
              5, 5, right),
        rcopy(wo_buf.at[1, pl.ds(HWO, HWO), :], wo_buf.at[2, pl.ds(HWO, HWO), :],
              7, 7, left),
    ]
    _wo_done = []

    def wo_ready():
        if not _wo_done:
            sends[2].wait_recv()
            sends[3].wait_recv()
            for r in fwds_wo:
                r.start()
            _wo_done.append(True)

    compute(wq_buf.at[0], wo_buf.at[0], (my - 1) % N_DEV, first=False,
            wo_wait=wo_ready)
    compute(wq_buf.at[1], wo_buf.at[1], (my + 1) % N_DEV, first=False,
            wo_wait=wo_ready)

    for r in fwds_wq:
        r.wait_recv()

    def wo_diag_ready():
        for r in fwds_wo:
            r.wait_recv()

    compute(wq_buf.at[2], wo_buf.at[2], (my + 2) % N_DEV, first=False,
            wo_wait=wo_diag_ready)

    for r in sends + fwds_wq + fwds_wo:
        r.wait_send()


def kernel(x, Wq, K_ext, V_ext, Wo):
    my = lax.axis_index("i")
    Kb = lax.dynamic_slice_in_dim(K_ext, my * B_LOC, B_LOC, axis=0)
    Vb = lax.dynamic_slice_in_dim(V_ext, my * B_LOC, B_LOC, axis=0)
    Kt = jnp.transpose(Kb.astype(jnp.bfloat16), (2, 0, 1, 3))
    Vt = jnp.transpose(Vb.astype(jnp.bfloat16), (2, 0, 1, 3))

    return pl.pallas_call(
        _body,
        out_shape=jax.ShapeDtypeStruct((B_LOC, SQ, D_MODEL), jnp.float32),
        in_specs=[pl.BlockSpec(memory_space=pltpu.VMEM)] * 5,
        out_specs=pl.BlockSpec(memory_space=pltpu.VMEM),
        scratch_shapes=[
            pltpu.VMEM((B_LOC, SQ, D_MODEL), jnp.bfloat16),
            pltpu.VMEM((D_MODEL, D_BLOCK), jnp.bfloat16),
            pltpu.VMEM((D_BLOCK, D_MODEL), jnp.bfloat16),
            pltpu.VMEM((N_DEV - 1, D_MODEL, D_BLOCK), jnp.bfloat16),
            pltpu.VMEM((N_DEV - 1, D_BLOCK, D_MODEL), jnp.bfloat16),
            pltpu.VMEM((SQ, D_BLOCK), jnp.bfloat16),
            pltpu.SemaphoreType.DMA((8,)),
            pltpu.SemaphoreType.DMA((8,)),
        ],
        compiler_params=_CompilerParams(
            collective_id=0, vmem_limit_bytes=100 * 1024 * 1024),
    )(x, Wq, Kt, Vt, Wo)
